# Initial kernel scaffold; baseline (speedup 1.0000x reference)
import jax
import jax.numpy as jnp
from jax.experimental import pallas as pl


def kernel(z, codebook):
    raise NotImplementedError("write your pallas kernel here")


# speedup vs baseline: 3.5525x; 3.5525x over previous
import jax
import jax.numpy as jnp
from jax import lax
from jax.experimental import pallas as pl
from jax.experimental.pallas import tpu as pltpu
from jax.experimental.pallas import tpu_sc as plsc

_N_E = 8192
_E_DIM = 32
_BETA = 0.25
_ROWS = 8192
_BLK = 256
_GRID = _ROWS // _BLK


def _tc_body(zb, cbt, zs, cs, idx_ref, loss_ref):
    scores = lax.dot_general(zb[...], cbt[...], (((1,), (0,)), ((), ())),
                             preferred_element_type=jnp.float32)
    d = (zs[...] + cs[...]) - 2.0 * scores
    mn = jnp.min(d, axis=1, keepdims=True)
    lanes = lax.broadcasted_iota(jnp.int32, d.shape, 1)
    idx = jnp.min(jnp.where(d == mn, lanes, jnp.int32(2**31 - 1)), axis=1)
    idx_ref[...] = idx[:, None]
    s = jnp.sum(mn, axis=0, keepdims=True)

    @pl.when(pl.program_id(0) == 0)
    def _init():
        loss_ref[...] = s

    @pl.when(pl.program_id(0) > 0)
    def _acc():
        loss_ref[...] += s


_tc_argmin = pl.pallas_call(
    _tc_body,
    grid=(_GRID,),
    in_specs=[
        pl.BlockSpec((_BLK, _E_DIM), lambda i: (i, 0)),
        pl.BlockSpec((_E_DIM, _N_E), lambda i: (0, 0)),
        pl.BlockSpec((_BLK, 1), lambda i: (i, 0)),
        pl.BlockSpec((1, _N_E), lambda i: (0, 0)),
    ],
    out_specs=[
        pl.BlockSpec((_BLK, 1), lambda i: (i, 0)),
        pl.BlockSpec((1, 1), lambda i: (0, 0)),
    ],
    out_shape=[
        jax.ShapeDtypeStruct((_ROWS, 1), jnp.int32),
        jax.ShapeDtypeStruct((1, 1), jnp.float32),
    ],
)


_PADW = 128


def _sc_gather_call(cb_pad, idx):
    info = plsc.get_sparse_core_info()
    nw = info.num_cores * info.num_subcores
    bpw = _ROWS // nw
    nc = info.num_cores

    def body(cb_hbm, idx_hbm, out_hbm, idx_v, rows_v, sem):
        wid = lax.axis_index("s") * nc + lax.axis_index("c")
        base = wid * bpw
        pltpu.sync_copy(idx_hbm.at[pl.ds(base, bpw)], idx_v)
        pltpu.async_copy(cb_hbm.at[idx_v], rows_v, sem).wait()
        pltpu.sync_copy(rows_v, out_hbm.at[pl.ds(base, bpw)])

    gather = pl.kernel(
        body,
        out_type=jax.ShapeDtypeStruct((_ROWS, _PADW), jnp.float32),
        mesh=plsc.VectorSubcoreMesh(core_axis_name="c", subcore_axis_name="s"),
        scratch_types=[
            pltpu.VMEM((bpw,), jnp.int32),
            pltpu.VMEM((bpw, _PADW), jnp.float32),
            pltpu.SemaphoreType.DMA,
        ],
    )
    return gather(cb_pad, idx)


def kernel(z, codebook):
    zp = jnp.transpose(z, (0, 2, 3, 1))
    z_flat = zp.reshape(-1, _E_DIM)
    zsum = jnp.sum(z_flat**2, axis=1, keepdims=True)
    csum = jnp.sum(codebook**2, axis=1).reshape(1, -1)
    z_bf = z_flat.astype(jnp.bfloat16)
    cbt_bf = codebook.astype(jnp.bfloat16).T
    cb_q = codebook.astype(jnp.bfloat16).astype(jnp.float32)
    cb_pad = jnp.pad(cb_q, ((0, 0), (0, _PADW - _E_DIM)))

    idx2d, loss_sum = _tc_argmin(z_bf, cbt_bf, zsum, csum)
    idx = idx2d.reshape(-1)
    zq_flat = _sc_gather_call(cb_pad, idx)[:, :_E_DIM]

    v = loss_sum[0, 0] / jnp.float32(_ROWS * _E_DIM)
    codebook_loss = v + v * jnp.float32(_BETA)
    z_q = zq_flat.reshape(zp.shape).transpose(0, 3, 1, 2)
    return (z_q, codebook_loss)


# speedup vs baseline: 3.8624x; 1.0872x over previous
import jax
import jax.numpy as jnp
from jax import lax
from jax.experimental import pallas as pl
from jax.experimental.pallas import tpu as pltpu
from jax.experimental.pallas import tpu_sc as plsc

_N_E = 8192
_E_DIM = 32
_BETA = 0.25
_ROWS = 8192
_BLK = 256
_GRID = _ROWS // _BLK


def _tc_body(zb, cbt, zs, cs, lanes, idx_ref, loss_ref):
    zt = jnp.transpose(zb[0], (1, 0)).astype(jnp.bfloat16)
    scores = lax.dot_general(zt, cbt[...], (((1,), (0,)), ((), ())),
                             preferred_element_type=jnp.float32)
    d = (zs[...] + cs[...]) - 2.0 * scores
    mn = jnp.min(d, axis=1, keepdims=True)
    idx = jnp.min(jnp.where(d == mn, lanes[...], jnp.float32(2**24)), axis=1)
    idx_ref[...] = idx[:, None].astype(jnp.int32)
    s = jnp.sum(mn, axis=0, keepdims=True)

    @pl.when(pl.program_id(0) == 0)
    def _init():
        loss_ref[...] = s

    @pl.when(pl.program_id(0) > 0)
    def _acc():
        loss_ref[...] += s


_tc_argmin = pl.pallas_call(
    _tc_body,
    grid=(_GRID,),
    in_specs=[
        pl.BlockSpec((1, _E_DIM, _BLK), lambda i: (i // 4, 0, i % 4)),
        pl.BlockSpec((_E_DIM, _N_E), lambda i: (0, 0)),
        pl.BlockSpec((_BLK, 1), lambda i: (i, 0)),
        pl.BlockSpec((1, _N_E), lambda i: (0, 0)),
        pl.BlockSpec((1, _N_E), lambda i: (0, 0)),
    ],
    out_specs=[
        pl.BlockSpec((_BLK, 1), lambda i: (i, 0)),
        pl.BlockSpec((1, 1), lambda i: (0, 0)),
    ],
    out_shape=[
        jax.ShapeDtypeStruct((_ROWS, 1), jnp.int32),
        jax.ShapeDtypeStruct((1, 1), jnp.float32),
    ],
)


_PADW = 128


def _sc_gather_call(cb_pad, idx):
    info = plsc.get_sparse_core_info()
    nw = info.num_cores * info.num_subcores
    bpw = _ROWS // nw
    nc = info.num_cores

    def body(cb_hbm, idx_hbm, out_hbm, idx_v, rows_v, sem):
        wid = lax.axis_index("s") * nc + lax.axis_index("c")
        base = wid * bpw
        pltpu.sync_copy(idx_hbm.at[pl.ds(base, bpw)], idx_v)
        pltpu.async_copy(cb_hbm.at[idx_v], rows_v, sem).wait()
        pltpu.sync_copy(rows_v, out_hbm.at[pl.ds(base, bpw)])

    gather = pl.kernel(
        body,
        out_type=jax.ShapeDtypeStruct((_ROWS, _PADW), jnp.float32),
        mesh=plsc.VectorSubcoreMesh(core_axis_name="c", subcore_axis_name="s"),
        scratch_types=[
            pltpu.VMEM((bpw,), jnp.int32),
            pltpu.VMEM((bpw, _PADW), jnp.float32),
            pltpu.SemaphoreType.DMA,
        ],
    )
    return gather(cb_pad, idx)


def kernel(z, codebook):
    zp = jnp.transpose(z, (0, 2, 3, 1))
    z_flat = zp.reshape(-1, _E_DIM)
    zsum = jnp.sum(z_flat**2, axis=1, keepdims=True)
    csum = jnp.sum(codebook**2, axis=1).reshape(1, -1)
    z3 = z.reshape(8, _E_DIM, 1024)
    cbt_bf = codebook.astype(jnp.bfloat16).T
    cb_q = codebook.astype(jnp.bfloat16).astype(jnp.float32)
    cb_pad = jnp.pad(cb_q, ((0, 0), (0, _PADW - _E_DIM)))

    lanes = jnp.arange(_N_E, dtype=jnp.float32).reshape(1, -1)
    idx2d, loss_sum = _tc_argmin(z3, cbt_bf, zsum, csum, lanes)
    idx = idx2d.reshape(-1)
    zq_flat = _sc_gather_call(cb_pad, idx)[:, :_E_DIM]

    v = loss_sum[0, 0] / jnp.float32(_ROWS * _E_DIM)
    codebook_loss = v + v * jnp.float32(_BETA)
    z_q = zq_flat.reshape(zp.shape).transpose(0, 3, 1, 2)
    return (z_q, codebook_loss)


# speedup vs baseline: 3.9543x; 1.0238x over previous
import jax
import jax.numpy as jnp
from jax import lax
from jax.experimental import pallas as pl
from jax.experimental.pallas import tpu as pltpu
from jax.experimental.pallas import tpu_sc as plsc

_N_E = 8192
_E_DIM = 32
_BETA = 0.25
_ROWS = 8192
_BLK = 512
_GRID = _ROWS // _BLK
_BPI = 1024 // _BLK


def _tc_body(zb, cbt, zs, cs, lanes, idx_ref, loss_ref):
    zt = jnp.transpose(zb[0], (1, 0)).astype(jnp.bfloat16)
    scores = lax.dot_general(zt, cbt[...], (((1,), (0,)), ((), ())),
                             preferred_element_type=jnp.float32)
    d = (zs[...] + cs[...]) - 2.0 * scores
    mn = jnp.min(d, axis=1, keepdims=True)
    idx = jnp.min(jnp.where(d == mn, lanes[...], jnp.float32(2**24)), axis=1)
    idx_ref[...] = idx[:, None].astype(jnp.int32)
    s = jnp.sum(mn, axis=0, keepdims=True)

    @pl.when(pl.program_id(0) == 0)
    def _init():
        loss_ref[...] = s

    @pl.when(pl.program_id(0) > 0)
    def _acc():
        loss_ref[...] += s


_tc_argmin = pl.pallas_call(
    _tc_body,
    grid=(_GRID,),
    in_specs=[
        pl.BlockSpec((1, _E_DIM, _BLK), lambda i: (i // _BPI, 0, i % _BPI)),
        pl.BlockSpec((_E_DIM, _N_E), lambda i: (0, 0)),
        pl.BlockSpec((_BLK, 1), lambda i: (i, 0)),
        pl.BlockSpec((1, _N_E), lambda i: (0, 0)),
        pl.BlockSpec((1, _N_E), lambda i: (0, 0)),
    ],
    out_specs=[
        pl.BlockSpec((_BLK, 1), lambda i: (i, 0)),
        pl.BlockSpec((1, 1), lambda i: (0, 0)),
    ],
    out_shape=[
        jax.ShapeDtypeStruct((_ROWS, 1), jnp.int32),
        jax.ShapeDtypeStruct((1, 1), jnp.float32),
    ],
)


_PADW = 128


_LANES = 16


def _sc_gather_call(cb_pad, idx):
    info = plsc.get_sparse_core_info()
    nw = info.num_cores * info.num_subcores
    bpw = _ROWS // nw
    nc = info.num_cores
    tpi = 1024 // bpw

    def body(cb_hbm, idx_hbm, out_hbm, idx_v, rows_v, outt_v, sem):
        wid = lax.axis_index("s") * nc + lax.axis_index("c")
        base = wid * bpw
        pltpu.sync_copy(idx_hbm.at[pl.ds(base, bpw)], idx_v)
        pltpu.async_copy(cb_hbm.at[idx_v], rows_v, sem).wait()
        for c in range(_E_DIM):
            cvec = jnp.full((_LANES,), c, dtype=jnp.int32)
            for p16 in range(bpw // _LANES):
                pidx = p16 * _LANES + lax.iota(jnp.int32, _LANES)
                outt_v[c, pl.ds(p16 * _LANES, _LANES)] = plsc.load_gather(
                    rows_v, [pidx, cvec])
        b = wid // tpi
        p0 = (wid % tpi) * bpw
        pltpu.sync_copy(outt_v, out_hbm.at[b, :, pl.ds(p0, bpw)])

    gather = pl.kernel(
        body,
        out_type=jax.ShapeDtypeStruct((8, _E_DIM, 1024), jnp.float32),
        mesh=plsc.VectorSubcoreMesh(core_axis_name="c", subcore_axis_name="s"),
        compiler_params=pltpu.CompilerParams(needs_layout_passes=False),
        scratch_types=[
            pltpu.VMEM((bpw,), jnp.int32),
            pltpu.VMEM((bpw, _PADW), jnp.float32),
            pltpu.VMEM((_E_DIM, bpw), jnp.float32),
            pltpu.SemaphoreType.DMA,
        ],
    )
    return gather(cb_pad, idx)


def kernel(z, codebook):
    zp = jnp.transpose(z, (0, 2, 3, 1))
    z_flat = zp.reshape(-1, _E_DIM)
    zsum = jnp.sum(z_flat**2, axis=1, keepdims=True)
    csum = jnp.sum(codebook**2, axis=1).reshape(1, -1)
    z3 = z.reshape(8, _E_DIM, 1024)
    cbt_bf = codebook.astype(jnp.bfloat16).T
    cb_q = codebook.astype(jnp.bfloat16).astype(jnp.float32)
    cb_pad = jnp.pad(cb_q, ((0, 0), (0, _PADW - _E_DIM)))

    lanes = jnp.arange(_N_E, dtype=jnp.float32).reshape(1, -1)
    idx2d, loss_sum = _tc_argmin(z3, cbt_bf, zsum, csum, lanes)
    idx = idx2d.reshape(-1)
    zq3 = _sc_gather_call(cb_pad, idx)

    v = loss_sum[0, 0] / jnp.float32(_ROWS * _E_DIM)
    codebook_loss = v + v * jnp.float32(_BETA)
    z_q = zq3.reshape(z.shape)
    return (z_q, codebook_loss)


# speedup vs baseline: 3.9988x; 1.0113x over previous
import jax
import jax.numpy as jnp
from jax import lax
from jax.experimental import pallas as pl
from jax.experimental.pallas import tpu as pltpu
from jax.experimental.pallas import tpu_sc as plsc

_N_E = 8192
_E_DIM = 32
_BETA = 0.25
_ROWS = 8192
_BLK = 512
_GRID = _ROWS // _BLK
_BPI = 1024 // _BLK


def _tc_body(zb, cbt, zs, cs, idx_ref, loss_ref):
    lanes = lax.broadcasted_iota(jnp.int32, (1, _N_E), 1).astype(jnp.float32)
    zt = zb[0].reshape(_E_DIM, _BLK).astype(jnp.bfloat16)
    scores = lax.dot_general(zt, cbt[...], (((0,), (0,)), ((), ())),
                             preferred_element_type=jnp.float32)
    d = (zs[...] + cs[...]) - 2.0 * scores
    mn = jnp.min(d, axis=1, keepdims=True)
    idx = jnp.min(jnp.where(d == mn, lanes, jnp.float32(2**24)), axis=1)
    idx_ref[...] = idx[:, None].astype(jnp.int32)
    s = jnp.sum(mn, axis=0, keepdims=True)

    @pl.when(pl.program_id(0) == 0)
    def _init():
        loss_ref[...] = s

    @pl.when(pl.program_id(0) > 0)
    def _acc():
        loss_ref[...] += s


_tc_argmin = pl.pallas_call(
    _tc_body,
    grid=(_GRID,),
    in_specs=[
        pl.BlockSpec((1, _E_DIM, _BLK // 32, 32),
                     lambda i: (i // _BPI, 0, i % _BPI, 0)),
        pl.BlockSpec((_E_DIM, _N_E), lambda i: (0, 0)),
        pl.BlockSpec((_BLK, 1), lambda i: (i, 0)),
        pl.BlockSpec((1, _N_E), lambda i: (0, 0)),
    ],
    out_specs=[
        pl.BlockSpec((_BLK, 1), lambda i: (i, 0)),
        pl.BlockSpec((1, 1), lambda i: (0, 0)),
    ],
    out_shape=[
        jax.ShapeDtypeStruct((_ROWS, 1), jnp.int32),
        jax.ShapeDtypeStruct((1, 1), jnp.float32),
    ],
)


_PADW = 128


_LANES = 16


def _sc_gather_call(cb_pad, idx):
    info = plsc.get_sparse_core_info()
    nw = info.num_cores * info.num_subcores
    bpw = _ROWS // nw
    nc = info.num_cores
    tpi = 1024 // bpw

    def body(cb_hbm, idx_hbm, out_hbm, idx_v, rows_v, outt_v, sem):
        wid = lax.axis_index("s") * nc + lax.axis_index("c")
        base = wid * bpw
        pltpu.sync_copy(idx_hbm.at[pl.ds(base, bpw)], idx_v)
        pltpu.async_copy(cb_hbm.at[idx_v], rows_v, sem).wait()
        for c in range(_E_DIM):
            cvec = jnp.full((_LANES,), c, dtype=jnp.int32)
            for p16 in range(bpw // _LANES):
                pidx = p16 * _LANES + lax.iota(jnp.int32, _LANES)
                outt_v[c, pl.ds(p16 * _LANES, _LANES)] = plsc.load_gather(
                    rows_v, [pidx, cvec])
        b = wid // tpi
        p0 = (wid % tpi) * bpw
        pltpu.sync_copy(outt_v, out_hbm.at[b, :, pl.ds(p0, bpw)])

    gather = pl.kernel(
        body,
        out_type=jax.ShapeDtypeStruct((8, _E_DIM, 1024), jnp.float32),
        mesh=plsc.VectorSubcoreMesh(core_axis_name="c", subcore_axis_name="s"),
        compiler_params=pltpu.CompilerParams(needs_layout_passes=False),
        scratch_types=[
            pltpu.VMEM((bpw,), jnp.int32),
            pltpu.VMEM((bpw, _PADW), jnp.float32),
            pltpu.VMEM((_E_DIM, bpw), jnp.float32),
            pltpu.SemaphoreType.DMA,
        ],
    )
    return gather(cb_pad, idx)


def kernel(z, codebook):
    zp = jnp.transpose(z, (0, 2, 3, 1))
    z_flat = zp.reshape(-1, _E_DIM)
    zsum = jnp.sum(z_flat**2, axis=1, keepdims=True)
    csum = jnp.sum(codebook**2, axis=1).reshape(1, -1)
    cbt_bf = codebook.astype(jnp.bfloat16).T
    cb_q = codebook.astype(jnp.bfloat16).astype(jnp.float32)
    cb_pad = jnp.pad(cb_q, ((0, 0), (0, _PADW - _E_DIM)))

    idx2d, loss_sum = _tc_argmin(z, cbt_bf, zsum, csum)
    idx = idx2d.reshape(-1)
    zq3 = _sc_gather_call(cb_pad, idx)

    v = loss_sum[0, 0] / jnp.float32(_ROWS * _E_DIM)
    codebook_loss = v + v * jnp.float32(_BETA)
    z_q = zq3.reshape(z.shape)
    return (z_q, codebook_loss)


# speedup vs baseline: 4.1019x; 1.0258x over previous
import jax
import jax.numpy as jnp
from jax import lax
from jax.experimental import pallas as pl
from jax.experimental.pallas import tpu as pltpu
from jax.experimental.pallas import tpu_sc as plsc

_N_E = 8192
_E_DIM = 32
_BETA = 0.25
_ROWS = 8192
_BLK = 512
_GRID = _ROWS // _BLK
_BPI = 1024 // _BLK


def _tc_body(zb, cbt, zs, cs, idx_ref, loss_ref):
    lanes = lax.broadcasted_iota(jnp.int32, (1, _N_E), 1).astype(jnp.float32)
    zt = zb[0].reshape(_E_DIM, _BLK).astype(jnp.bfloat16)
    scores = lax.dot_general(zt, cbt[...], (((0,), (0,)), ((), ())),
                             preferred_element_type=jnp.float32)
    d = (zs[...] + cs[...]) - 2.0 * scores
    mn = jnp.min(d, axis=1, keepdims=True)
    idx = jnp.min(jnp.where(d == mn, lanes, jnp.float32(2**24)), axis=1)
    idx_ref[...] = idx[:, None].astype(jnp.int32)
    s = jnp.sum(mn, axis=0, keepdims=True)

    @pl.when(pl.program_id(0) == 0)
    def _init():
        loss_ref[...] = s

    @pl.when(pl.program_id(0) > 0)
    def _acc():
        loss_ref[...] += s


_tc_argmin = pl.pallas_call(
    _tc_body,
    grid=(_GRID,),
    in_specs=[
        pl.BlockSpec((1, _E_DIM, _BLK // 32, 32),
                     lambda i: (i // _BPI, 0, i % _BPI, 0)),
        pl.BlockSpec((_E_DIM, _N_E), lambda i: (0, 0)),
        pl.BlockSpec((_BLK, 1), lambda i: (i, 0)),
        pl.BlockSpec((1, _N_E), lambda i: (0, 0)),
    ],
    out_specs=[
        pl.BlockSpec((_BLK, 1), lambda i: (i, 0)),
        pl.BlockSpec((1, 1), lambda i: (0, 0)),
    ],
    out_shape=[
        jax.ShapeDtypeStruct((_ROWS, 1), jnp.int32),
        jax.ShapeDtypeStruct((1, 1), jnp.float32),
    ],
)


_PADW = 128


_LANES = 16


def _sc_gather_call(cb_pad, idx):
    info = plsc.get_sparse_core_info()
    nw = info.num_cores * info.num_subcores
    bpw = _ROWS // nw
    nc = info.num_cores
    tpi = 1024 // bpw

    def body(cb_hbm, idx_hbm, out_hbm, idx_v, rows_v, outt_v, sem):
        wid = lax.axis_index("s") * nc + lax.axis_index("c")
        base = wid * bpw
        pltpu.sync_copy(idx_hbm.at[pl.ds(base, bpw)], idx_v)
        pltpu.async_copy(cb_hbm.at[idx_v], rows_v, sem).wait()
        hpt = bpw // 32
        for c in range(_E_DIM):
            cvec = jnp.full((_LANES,), c, dtype=jnp.int32)
            for p16 in range(bpw // _LANES):
                pidx = p16 * _LANES + lax.iota(jnp.int32, _LANES)
                outt_v[c, p16 // 2, pl.ds((p16 % 2) * _LANES, _LANES)] = (
                    plsc.load_gather(rows_v, [pidx, cvec]))
        b = wid // tpi
        h0 = (wid % tpi) * hpt
        pltpu.sync_copy(outt_v, out_hbm.at[b, :, pl.ds(h0, hpt), :])

    gather = pl.kernel(
        body,
        out_type=jax.ShapeDtypeStruct((8, _E_DIM, 32, 32), jnp.float32),
        mesh=plsc.VectorSubcoreMesh(core_axis_name="c", subcore_axis_name="s"),
        compiler_params=pltpu.CompilerParams(needs_layout_passes=False),
        scratch_types=[
            pltpu.VMEM((bpw,), jnp.int32),
            pltpu.VMEM((bpw, _PADW), jnp.float32),
            pltpu.VMEM((_E_DIM, bpw // 32, 32), jnp.float32),
            pltpu.SemaphoreType.DMA,
        ],
    )
    return gather(cb_pad, idx)


def kernel(z, codebook):
    zp = jnp.transpose(z, (0, 2, 3, 1))
    z_flat = zp.reshape(-1, _E_DIM)
    zsum = jnp.sum(z_flat**2, axis=1, keepdims=True)
    csum = jnp.sum(codebook**2, axis=1).reshape(1, -1)
    cbt_bf = codebook.astype(jnp.bfloat16).T
    cb_q = codebook.astype(jnp.bfloat16).astype(jnp.float32)
    cb_pad = jnp.pad(cb_q, ((0, 0), (0, _PADW - _E_DIM)))

    idx2d, loss_sum = _tc_argmin(z, cbt_bf, zsum, csum)
    idx = idx2d.reshape(-1)
    z_q = _sc_gather_call(cb_pad, idx)

    v = loss_sum[0, 0] / jnp.float32(_ROWS * _E_DIM)
    codebook_loss = v + v * jnp.float32(_BETA)
    return (z_q, codebook_loss)


# speedup vs baseline: 4.2632x; 1.0393x over previous
import jax
import jax.numpy as jnp
from jax import lax
from jax.experimental import pallas as pl
from jax.experimental.pallas import tpu as pltpu
from jax.experimental.pallas import tpu_sc as plsc

_N_E = 8192
_E_DIM = 32
_BETA = 0.25
_ROWS = 8192
_BLK = 512
_GRID = _ROWS // _BLK
_BPI = 1024 // _BLK


def _tc_body(zb, cbt, zs, cs, idx_ref, loss_ref):
    lanes = lax.broadcasted_iota(jnp.int32, (1, _N_E), 1).astype(jnp.float32)
    zt = zb[0].reshape(_E_DIM, _BLK).astype(jnp.bfloat16)
    scores = lax.dot_general(zt, cbt[...], (((0,), (0,)), ((), ())),
                             preferred_element_type=jnp.float32)
    zcol = jnp.transpose(zs[...], (1, 0))
    d = (zcol + cs[...]) - 2.0 * scores
    mn = jnp.min(d, axis=1, keepdims=True)
    idx = jnp.min(jnp.where(d == mn, lanes, jnp.float32(2**24)), axis=1)
    idx_ref[...] = idx[None, :].astype(jnp.int32)
    s = jnp.sum(mn, axis=0, keepdims=True)

    @pl.when(pl.program_id(0) == 0)
    def _init():
        loss_ref[...] = s

    @pl.when(pl.program_id(0) > 0)
    def _acc():
        loss_ref[...] += s


_tc_argmin = pl.pallas_call(
    _tc_body,
    grid=(_GRID,),
    in_specs=[
        pl.BlockSpec((1, _E_DIM, _BLK // 32, 32),
                     lambda i: (i // _BPI, 0, i % _BPI, 0)),
        pl.BlockSpec((_E_DIM, _N_E), lambda i: (0, 0)),
        pl.BlockSpec((1, _BLK), lambda i: (0, i)),
        pl.BlockSpec((1, _N_E), lambda i: (0, 0)),
    ],
    out_specs=[
        pl.BlockSpec((1, _BLK), lambda i: (0, i)),
        pl.BlockSpec((1, 1), lambda i: (0, 0)),
    ],
    out_shape=[
        jax.ShapeDtypeStruct((1, _ROWS), jnp.int32),
        jax.ShapeDtypeStruct((1, 1), jnp.float32),
    ],
)


_PADW = 128


_LANES = 16


def _sc_gather_call(cb_pad, idx):
    info = plsc.get_sparse_core_info()
    nw = info.num_cores * info.num_subcores
    bpw = _ROWS // nw
    nc = info.num_cores
    tpi = 1024 // bpw

    def body(cb_hbm, idx_hbm, out_hbm, idx_v, rows_v, outt_v, sem):
        wid = lax.axis_index("s") * nc + lax.axis_index("c")
        base = wid * bpw
        pltpu.sync_copy(idx_hbm.at[0, pl.ds(base, bpw)], idx_v)
        pltpu.async_copy(cb_hbm.at[idx_v], rows_v, sem).wait()
        hpt = bpw // 32

        @plsc.parallel_loop(0, bpw // _LANES)
        def _t(p16):
            pidx = p16 * _LANES + lax.iota(jnp.int32, _LANES)
            h = p16 // 2
            w0 = (p16 % 2) * _LANES
            for c in range(_E_DIM):
                cvec = jnp.full((_LANES,), c, dtype=jnp.int32)
                outt_v[c, h, pl.ds(w0, _LANES)] = plsc.load_gather(
                    rows_v, [pidx, cvec])

        b = wid // tpi
        h0 = (wid % tpi) * hpt
        pltpu.sync_copy(outt_v, out_hbm.at[b, :, pl.ds(h0, hpt), :])

    gather = pl.kernel(
        body,
        out_type=jax.ShapeDtypeStruct((8, _E_DIM, 32, 32), jnp.float32),
        mesh=plsc.VectorSubcoreMesh(core_axis_name="c", subcore_axis_name="s"),
        compiler_params=pltpu.CompilerParams(needs_layout_passes=False),
        scratch_types=[
            pltpu.VMEM((bpw,), jnp.int32),
            pltpu.VMEM((bpw, _PADW), jnp.float32),
            pltpu.VMEM((_E_DIM, bpw // 32, 32), jnp.float32),
            pltpu.SemaphoreType.DMA,
        ],
    )
    return gather(cb_pad, idx)


def kernel(z, codebook):
    zp = jnp.transpose(z, (0, 2, 3, 1))
    z_flat = zp.reshape(-1, _E_DIM)
    zsum = jnp.sum(z_flat**2, axis=1).reshape(1, -1)
    csum = jnp.sum(codebook**2, axis=1).reshape(1, -1)
    cbt_bf = codebook.astype(jnp.bfloat16).T
    cb_q = codebook.astype(jnp.bfloat16).astype(jnp.float32)
    cb_pad = jnp.pad(cb_q, ((0, 0), (0, _PADW - _E_DIM)))

    idx2d, loss_sum = _tc_argmin(z, cbt_bf, zsum, csum)
    z_q = _sc_gather_call(cb_pad, idx2d)

    v = loss_sum[0, 0] / jnp.float32(_ROWS * _E_DIM)
    codebook_loss = v + v * jnp.float32(_BETA)
    return (z_q, codebook_loss)
